# Initial kernel scaffold; baseline (speedup 1.0000x reference)
import jax
import jax.numpy as jnp
from jax.experimental import pallas as pl


def kernel(x, emb_table, W, b):
    raise NotImplementedError("write your pallas kernel here")


# speedup vs baseline: 30.1054x; 30.1054x over previous
import functools

import jax
import jax.numpy as jnp
from jax import lax
from jax.experimental import pallas as pl
from jax.experimental.pallas import tpu as pltpu
from jax.experimental.pallas import tpu_sc as plsc

VOCAB = 1000000
HIDDEN = 128
OUT = 2
BATCH = 4096
SEQ = 200

NTOK = BATCH * SEQ
NW = 32
BPW = NTOK // NW
CHUNK = 128
NCHUNK = BPW // CHUNK

PROJ_BLK = 8192


def _proj_body(table_ref, w_ref, b_ref, out_ref):
    acc = lax.dot_general(
        w_ref[...], table_ref[...],
        dimension_numbers=(((0,), (1,)), ((), ())),
        preferred_element_type=jnp.float32,
    )
    out_ref[...] = acc + b_ref[...]


def _project_table(emb_table, W, b):
    return pl.pallas_call(
        _proj_body,
        grid=(pl.cdiv(VOCAB, PROJ_BLK),),
        in_specs=[
            pl.BlockSpec((PROJ_BLK, HIDDEN), lambda i: (i, 0)),
            pl.BlockSpec((HIDDEN, OUT), lambda i: (0, 0)),
            pl.BlockSpec((OUT, 1), lambda i: (0, 0)),
        ],
        out_specs=pl.BlockSpec((OUT, PROJ_BLK), lambda i: (0, i)),
        out_shape=jax.ShapeDtypeStruct((OUT, VOCAB), jnp.float32),
    )(emb_table, W, b.reshape(OUT, 1))


def _gather_body(p0_hbm, p1_hbm, idx_hbm, o0_hbm, o1_hbm, idx_v, r0_v, r1_v, sem):
    c = lax.axis_index("c")
    s = lax.axis_index("s")
    wid = s * 2 + c
    base = wid * BPW
    pltpu.sync_copy(idx_hbm.at[pl.ds(base, BPW)], idx_v)

    def chunk(g, carry):
        off = g * CHUNK
        idx_slice = idx_v.at[pl.ds(off, CHUNK)]
        cp0 = pltpu.async_copy(p0_hbm.at[idx_slice], r0_v.at[pl.ds(off, CHUNK)], sem)
        cp1 = pltpu.async_copy(p1_hbm.at[idx_slice], r1_v.at[pl.ds(off, CHUNK)], sem)
        cp0.wait()
        cp1.wait()
        return carry

    lax.fori_loop(0, NCHUNK, chunk, 0)
    pltpu.sync_copy(r0_v, o0_hbm.at[pl.ds(base, BPW)])
    pltpu.sync_copy(r1_v, o1_hbm.at[pl.ds(base, BPW)])


_gather = functools.partial(
    pl.kernel,
    mesh=plsc.VectorSubcoreMesh(core_axis_name="c", subcore_axis_name="s"),
    out_type=(
        jax.ShapeDtypeStruct((NTOK,), jnp.float32),
        jax.ShapeDtypeStruct((NTOK,), jnp.float32),
    ),
    scratch_types=[
        pltpu.VMEM((BPW,), jnp.int32),
        pltpu.VMEM((BPW,), jnp.float32),
        pltpu.VMEM((BPW,), jnp.float32),
        pltpu.SemaphoreType.DMA,
    ],
)(_gather_body)


@jax.jit
def kernel(x, emb_table, W, b):
    proj = _project_table(emb_table, W, b)
    o0, o1 = _gather(proj[0], proj[1], x.reshape(NTOK))
    return jnp.stack([o0, o1], axis=-1).reshape(BATCH, SEQ, OUT)


# speedup vs baseline: 44.6849x; 1.4843x over previous
import functools

import jax
import jax.numpy as jnp
from jax import lax
from jax.experimental import pallas as pl
from jax.experimental.pallas import tpu as pltpu
from jax.experimental.pallas import tpu_sc as plsc

VOCAB = 1000000
HIDDEN = 128
OUT = 2
BATCH = 4096
SEQ = 200

NTOK = BATCH * SEQ
NW = 32
BPW = NTOK // NW
CHUNK = 128
NCHUNK = BPW // CHUNK

PROJ_BLK = 16384


def _proj_body(table_ref, w_ref, b_ref, out_ref):
    acc = lax.dot_general(
        w_ref[...], table_ref[...],
        dimension_numbers=(((0,), (1,)), ((), ())),
        preferred_element_type=jnp.float32,
    )
    out_ref[...] = acc + b_ref[...]


def _project_table(emb_table, W, b):
    return pl.pallas_call(
        _proj_body,
        grid=(pl.cdiv(VOCAB, PROJ_BLK),),
        in_specs=[
            pl.BlockSpec((PROJ_BLK, HIDDEN), lambda i: (i, 0)),
            pl.BlockSpec((HIDDEN, OUT), lambda i: (0, 0)),
            pl.BlockSpec((OUT, 1), lambda i: (0, 0)),
        ],
        out_specs=pl.BlockSpec((OUT, PROJ_BLK), lambda i: (0, i)),
        out_shape=jax.ShapeDtypeStruct((OUT, VOCAB), jnp.float32),
    )(emb_table, W, b.reshape(OUT, 1))


def _gather_body(p0_hbm, p1_hbm, idx_hbm, o0_hbm, o1_hbm, idx_v, r0_v, r1_v, sem):
    c = lax.axis_index("c")
    s = lax.axis_index("s")
    wid = s * 2 + c
    base = wid * BPW
    pltpu.sync_copy(idx_hbm.at[pl.ds(base, BPW)], idx_v)

    def chunk(g, carry):
        off = g * CHUNK
        idx_slice = idx_v.at[pl.ds(off, CHUNK)]
        pltpu.async_copy(p0_hbm.at[idx_slice], r0_v.at[pl.ds(off, CHUNK)], sem)
        pltpu.async_copy(p1_hbm.at[idx_slice], r1_v.at[pl.ds(off, CHUNK)], sem)
        return carry

    lax.fori_loop(0, NCHUNK, chunk, 0)
    pltpu.make_async_copy(o0_hbm.at[pl.ds(base, BPW)], r0_v, sem).wait()
    pltpu.make_async_copy(o1_hbm.at[pl.ds(base, BPW)], r1_v, sem).wait()
    pltpu.sync_copy(r0_v, o0_hbm.at[pl.ds(base, BPW)])
    pltpu.sync_copy(r1_v, o1_hbm.at[pl.ds(base, BPW)])


_gather = functools.partial(
    pl.kernel,
    mesh=plsc.VectorSubcoreMesh(core_axis_name="c", subcore_axis_name="s"),
    out_type=(
        jax.ShapeDtypeStruct((NTOK,), jnp.float32),
        jax.ShapeDtypeStruct((NTOK,), jnp.float32),
    ),
    scratch_types=[
        pltpu.VMEM((BPW,), jnp.int32),
        pltpu.VMEM((BPW,), jnp.float32),
        pltpu.VMEM((BPW,), jnp.float32),
        pltpu.SemaphoreType.DMA,
    ],
)(_gather_body)


@jax.jit
def kernel(x, emb_table, W, b):
    proj = _project_table(emb_table, W, b)
    o0, o1 = _gather(proj[0], proj[1], x.reshape(NTOK))
    return jnp.stack([o0, o1], axis=-1).reshape(BATCH, SEQ, OUT)
